# baseline (speedup 1.0000x reference)
import jax
import jax.numpy as jnp
from jax.experimental import pallas as pl

NUM_VIEWS = 12
PROJECTION_DIM = 512
BATCH = 4096

BLOCK_B = 64


def _body(x_ref, p_ref, o_ref):
    o_ref[...] = x_ref[...] + p_ref[...]


def kernel(onedimage, pos_table):
    out = pl.pallas_call(
        _body,
        grid=(BATCH // BLOCK_B,),
        in_specs=[
            pl.BlockSpec((BLOCK_B, NUM_VIEWS, PROJECTION_DIM), lambda i: (i, 0, 0)),
            pl.BlockSpec((NUM_VIEWS, PROJECTION_DIM), lambda i: (0, 0)),
        ],
        out_specs=pl.BlockSpec((BLOCK_B, NUM_VIEWS, PROJECTION_DIM), lambda i: (i, 0, 0)),
        out_shape=jax.ShapeDtypeStruct((BATCH, NUM_VIEWS, PROJECTION_DIM), jnp.float32),
    )(onedimage, pos_table)
    return out


# speedup vs baseline: 1.0274x; 1.0274x over previous
import jax
import jax.numpy as jnp
from jax.experimental import pallas as pl
from jax.experimental.pallas import tpu as pltpu

NUM_VIEWS = 12
PROJECTION_DIM = 512
BATCH = 4096

BLK = 64
NBUF = 8
NSTEPS = BATCH // BLK


def _body(x_hbm, p_ref, o_hbm, xbuf, obuf, insem, outsem):
    i = pl.program_id(0)

    def in_copy(idx, slot):
        return pltpu.make_async_copy(
            x_hbm.at[pl.ds(idx * BLK, BLK)], xbuf.at[slot], insem.at[slot]
        )

    def out_copy(idx, slot):
        return pltpu.make_async_copy(
            obuf.at[slot], o_hbm.at[pl.ds(idx * BLK, BLK)], outsem.at[slot]
        )

    @pl.when(i == 0)
    def _prologue():
        for s in range(NBUF):
            in_copy(s, s).start()

    slot = jax.lax.rem(i, NBUF)
    in_copy(i, slot).wait()

    @pl.when(i >= NBUF)
    def _wait_out():
        out_copy(i - NBUF, slot).wait()

    obuf[slot] = xbuf[slot] + p_ref[...][None]

    out_copy(i, slot).start()

    @pl.when(i + NBUF < NSTEPS)
    def _next_in():
        in_copy(i + NBUF, slot).start()

    @pl.when(i == NSTEPS - 1)
    def _epilogue():
        for k in range(NBUF):
            idx = NSTEPS - NBUF + k
            out_copy(idx, idx % NBUF).wait()


def kernel(onedimage, pos_table):
    out = pl.pallas_call(
        _body,
        grid=(NSTEPS,),
        in_specs=[
            pl.BlockSpec(memory_space=pl.ANY),
            pl.BlockSpec(memory_space=pltpu.VMEM),
        ],
        out_specs=pl.BlockSpec(memory_space=pl.ANY),
        out_shape=jax.ShapeDtypeStruct((BATCH, NUM_VIEWS, PROJECTION_DIM), jnp.float32),
        scratch_shapes=[
            pltpu.VMEM((NBUF, BLK, NUM_VIEWS, PROJECTION_DIM), jnp.float32),
            pltpu.VMEM((NBUF, BLK, NUM_VIEWS, PROJECTION_DIM), jnp.float32),
            pltpu.SemaphoreType.DMA((NBUF,)),
            pltpu.SemaphoreType.DMA((NBUF,)),
        ],
    )(onedimage, pos_table)
    return out


# speedup vs baseline: 1.0277x; 1.0003x over previous
import jax
import jax.numpy as jnp
from jax import lax
from jax.experimental import pallas as pl
from jax.experimental.pallas import tpu as pltpu

NUM_VIEWS = 12
PROJECTION_DIM = 512
BATCH = 4096

BLK = 32
NBUF = 8
NJ = BATCH // (BLK * NBUF)


def _body(x_hbm, p_ref, o_hbm, xbuf, obuf, insem, outsem):
    j = pl.program_id(0)
    p = lax.rem(j, 2)

    def in_copy(jj, s):
        c = jj * NBUF + s
        return pltpu.make_async_copy(
            x_hbm.at[pl.ds(c * BLK, BLK)],
            xbuf.at[lax.rem(jj, 2), s],
            insem.at[lax.rem(jj, 2), s],
        )

    def out_copy(jj, s):
        c = jj * NBUF + s
        return pltpu.make_async_copy(
            obuf.at[lax.rem(jj, 2), s],
            o_hbm.at[pl.ds(c * BLK, BLK)],
            outsem.at[lax.rem(jj, 2), s],
        )

    @pl.when(j == 0)
    def _prologue():
        for s in range(NBUF):
            in_copy(0, s).start()

    @pl.when(j + 1 < NJ)
    def _prefetch():
        for s in range(NBUF):
            in_copy(j + 1, s).start()

    for s in range(NBUF):
        in_copy(j, s).wait()

    @pl.when(j >= 2)
    def _drain_old():
        for s in range(NBUF):
            out_copy(j - 2, s).wait()

    pos = p_ref[...][None]
    for s in range(NBUF):
        obuf[p, s] = xbuf[p, s] + pos

    for s in range(NBUF):
        out_copy(j, s).start()

    @pl.when(j == NJ - 1)
    def _epilogue():
        for s in range(NBUF):
            out_copy(j - 1, s).wait()
        for s in range(NBUF):
            out_copy(j, s).wait()


def kernel(onedimage, pos_table):
    out = pl.pallas_call(
        _body,
        grid=(NJ,),
        in_specs=[
            pl.BlockSpec(memory_space=pl.ANY),
            pl.BlockSpec(memory_space=pltpu.VMEM),
        ],
        out_specs=pl.BlockSpec(memory_space=pl.ANY),
        out_shape=jax.ShapeDtypeStruct((BATCH, NUM_VIEWS, PROJECTION_DIM), jnp.float32),
        scratch_shapes=[
            pltpu.VMEM((2, NBUF, BLK, NUM_VIEWS, PROJECTION_DIM), jnp.float32),
            pltpu.VMEM((2, NBUF, BLK, NUM_VIEWS, PROJECTION_DIM), jnp.float32),
            pltpu.SemaphoreType.DMA((2, NBUF)),
            pltpu.SemaphoreType.DMA((2, NBUF)),
        ],
    )(onedimage, pos_table)
    return out


# speedup vs baseline: 2.7157x; 2.6424x over previous
import jax
import jax.numpy as jnp
from jax.experimental import pallas as pl

NUM_VIEWS = 12
PROJECTION_DIM = 512
BATCH = 4096

BLK = 512


def _body(x_ref, p_ref, o_ref):
    o_ref[...] = x_ref[...] + p_ref[...]


def kernel(onedimage, pos_table):
    xt = jnp.transpose(onedimage, (1, 0, 2))
    p3 = pos_table.reshape(NUM_VIEWS, 1, PROJECTION_DIM)
    out_t = pl.pallas_call(
        _body,
        grid=(NUM_VIEWS, BATCH // BLK),
        in_specs=[
            pl.BlockSpec((1, BLK, PROJECTION_DIM), lambda v, i: (v, i, 0)),
            pl.BlockSpec((1, 1, PROJECTION_DIM), lambda v, i: (v, 0, 0)),
        ],
        out_specs=pl.BlockSpec((1, BLK, PROJECTION_DIM), lambda v, i: (v, i, 0)),
        out_shape=jax.ShapeDtypeStruct((NUM_VIEWS, BATCH, PROJECTION_DIM), jnp.float32),
    )(xt, p3)
    return jnp.transpose(out_t, (1, 0, 2))


# speedup vs baseline: 4.1180x; 1.5164x over previous
import jax
import jax.numpy as jnp
from jax.experimental import pallas as pl

NUM_VIEWS = 12
PROJECTION_DIM = 512
BATCH = 4096

BLK = 2048


def _body(x_ref, p_ref, o_ref):
    o_ref[...] = x_ref[...] + p_ref[...]


def kernel(onedimage, pos_table):
    xt = jnp.transpose(onedimage, (1, 0, 2))
    p3 = pos_table.reshape(NUM_VIEWS, 1, PROJECTION_DIM)
    out_t = pl.pallas_call(
        _body,
        grid=(NUM_VIEWS, BATCH // BLK),
        in_specs=[
            pl.BlockSpec((1, BLK, PROJECTION_DIM), lambda v, i: (v, i, 0)),
            pl.BlockSpec((1, 1, PROJECTION_DIM), lambda v, i: (v, 0, 0)),
        ],
        out_specs=pl.BlockSpec((1, BLK, PROJECTION_DIM), lambda v, i: (v, i, 0)),
        out_shape=jax.ShapeDtypeStruct((NUM_VIEWS, BATCH, PROJECTION_DIM), jnp.float32),
    )(xt, p3)
    return jnp.transpose(out_t, (1, 0, 2))


# speedup vs baseline: 4.2291x; 1.0270x over previous
import jax
import jax.numpy as jnp
from jax.experimental import pallas as pl

NUM_VIEWS = 12
PROJECTION_DIM = 512
BATCH = 4096

BLK = 4096


def _body(x_ref, p_ref, o_ref):
    o_ref[...] = x_ref[...] + p_ref[...]


def kernel(onedimage, pos_table):
    xt = jnp.transpose(onedimage, (1, 0, 2))
    p3 = pos_table.reshape(NUM_VIEWS, 1, PROJECTION_DIM)
    out_t = pl.pallas_call(
        _body,
        grid=(NUM_VIEWS, BATCH // BLK),
        in_specs=[
            pl.BlockSpec((1, BLK, PROJECTION_DIM), lambda v, i: (v, i, 0)),
            pl.BlockSpec((1, 1, PROJECTION_DIM), lambda v, i: (v, 0, 0)),
        ],
        out_specs=pl.BlockSpec((1, BLK, PROJECTION_DIM), lambda v, i: (v, i, 0)),
        out_shape=jax.ShapeDtypeStruct((NUM_VIEWS, BATCH, PROJECTION_DIM), jnp.float32),
    )(xt, p3)
    return jnp.transpose(out_t, (1, 0, 2))
